# baseline (speedup 1.0000x reference)
import jax
import jax.numpy as jnp
from jax.experimental import pallas as pl

_N = 100000
_D = 128
_BLOCK = 10000


def _sigmoid_block(p_ref, z_ref):
    z_ref[...] = 0.5 * jnp.tanh(p_ref[...] * 0.5) + 0.5


def kernel(P, test):
    return pl.pallas_call(
        _sigmoid_block,
        grid=(_N // _BLOCK,),
        in_specs=[pl.BlockSpec((_BLOCK, _D), lambda i: (i, 0))],
        out_specs=pl.BlockSpec((_BLOCK, _D), lambda i: (i, 0)),
        out_shape=jax.ShapeDtypeStruct((_N, _D), jnp.float32),
    )(P)


# speedup vs baseline: 1.0181x; 1.0181x over previous
import jax
import jax.numpy as jnp
from jax.experimental import pallas as pl
from jax.experimental.pallas import tpu as pltpu

_N = 100000
_D = 128
_CH = 10000
_NCH = _N // _CH


def _body(p_hbm, z_hbm, inb, outb, lsem, ssem):
    def load(c, slot):
        pltpu.make_async_copy(
            p_hbm.at[pl.ds(c * _CH, _CH)], inb.at[slot], lsem.at[slot]
        ).start()

    def wait_load(c, slot):
        pltpu.make_async_copy(
            p_hbm.at[pl.ds(c * _CH, _CH)], inb.at[slot], lsem.at[slot]
        ).wait()

    def store(c, slot):
        pltpu.make_async_copy(
            outb.at[slot], z_hbm.at[pl.ds(c * _CH, _CH)], ssem.at[slot]
        ).start()

    def wait_store(c, slot):
        pltpu.make_async_copy(
            outb.at[slot], z_hbm.at[pl.ds(c * _CH, _CH)], ssem.at[slot]
        ).wait()

    load(0, 0)
    for c in range(_NCH):
        s = c % 2
        if c + 1 < _NCH:
            load(c + 1, (c + 1) % 2)
        wait_load(c, s)
        if c >= 2:
            wait_store(c - 2, s)
        outb[s] = 0.5 * jnp.tanh(inb[s] * 0.5) + 0.5
        store(c, s)
    wait_store(_NCH - 2, (_NCH - 2) % 2)
    wait_store(_NCH - 1, (_NCH - 1) % 2)


def kernel(P, test):
    return pl.pallas_call(
        _body,
        in_specs=[pl.BlockSpec(memory_space=pl.ANY)],
        out_specs=pl.BlockSpec(memory_space=pl.ANY),
        out_shape=jax.ShapeDtypeStruct((_N, _D), jnp.float32),
        scratch_shapes=[
            pltpu.VMEM((2, _CH, _D), jnp.float32),
            pltpu.VMEM((2, _CH, _D), jnp.float32),
            pltpu.SemaphoreType.DMA((2,)),
            pltpu.SemaphoreType.DMA((2,)),
        ],
    )(P)


# speedup vs baseline: 1.0691x; 1.0501x over previous
import jax
import jax.numpy as jnp
from jax.experimental import pallas as pl
from jax.experimental.pallas import tpu as pltpu

_N = 100000
_D = 128
_CH = 25000
_NCH = _N // _CH


def _body(p_hbm, z_hbm, inb, outb, lsem, ssem):
    def load(c, slot):
        pltpu.make_async_copy(
            p_hbm.at[pl.ds(c * _CH, _CH)], inb.at[slot], lsem.at[slot]
        ).start()

    def wait_load(c, slot):
        pltpu.make_async_copy(
            p_hbm.at[pl.ds(c * _CH, _CH)], inb.at[slot], lsem.at[slot]
        ).wait()

    def store(c, slot):
        pltpu.make_async_copy(
            outb.at[slot], z_hbm.at[pl.ds(c * _CH, _CH)], ssem.at[slot]
        ).start()

    def wait_store(c, slot):
        pltpu.make_async_copy(
            outb.at[slot], z_hbm.at[pl.ds(c * _CH, _CH)], ssem.at[slot]
        ).wait()

    load(0, 0)
    for c in range(_NCH):
        s = c % 2
        if c + 1 < _NCH:
            load(c + 1, (c + 1) % 2)
        wait_load(c, s)
        if c >= 2:
            wait_store(c - 2, s)
        outb[s] = 0.5 * jnp.tanh(inb[s] * 0.5) + 0.5
        store(c, s)
    wait_store(_NCH - 2, (_NCH - 2) % 2)
    wait_store(_NCH - 1, (_NCH - 1) % 2)


def kernel(P, test):
    return pl.pallas_call(
        _body,
        in_specs=[pl.BlockSpec(memory_space=pl.ANY)],
        out_specs=pl.BlockSpec(memory_space=pl.ANY),
        out_shape=jax.ShapeDtypeStruct((_N, _D), jnp.float32),
        scratch_shapes=[
            pltpu.VMEM((2, _CH, _D), jnp.float32),
            pltpu.VMEM((2, _CH, _D), jnp.float32),
            pltpu.SemaphoreType.DMA((2,)),
            pltpu.SemaphoreType.DMA((2,)),
        ],
    )(P)


# speedup vs baseline: 1.0949x; 1.0241x over previous
import jax
import jax.numpy as jnp
from jax.experimental import pallas as pl
from jax.experimental.pallas import tpu as pltpu

_N = 100000
_D = 128
_CH = 50000
_NCH = _N // _CH


def _body(p_hbm, z_hbm, buf, lsem, ssem):
    def load(c):
        pltpu.make_async_copy(
            p_hbm.at[pl.ds(c * _CH, _CH)], buf.at[c], lsem.at[c]
        ).start()

    def wait_load(c):
        pltpu.make_async_copy(
            p_hbm.at[pl.ds(c * _CH, _CH)], buf.at[c], lsem.at[c]
        ).wait()

    def store(c):
        pltpu.make_async_copy(
            buf.at[c], z_hbm.at[pl.ds(c * _CH, _CH)], ssem.at[c]
        ).start()

    def wait_store(c):
        pltpu.make_async_copy(
            buf.at[c], z_hbm.at[pl.ds(c * _CH, _CH)], ssem.at[c]
        ).wait()

    load(0)
    load(1)
    for c in range(_NCH):
        wait_load(c)
        buf[c] = 0.5 * jnp.tanh(buf[c] * 0.5) + 0.5
        store(c)
    wait_store(0)
    wait_store(1)


def kernel(P, test):
    return pl.pallas_call(
        _body,
        in_specs=[pl.BlockSpec(memory_space=pl.ANY)],
        out_specs=pl.BlockSpec(memory_space=pl.ANY),
        out_shape=jax.ShapeDtypeStruct((_N, _D), jnp.float32),
        scratch_shapes=[
            pltpu.VMEM((_NCH, _CH, _D), jnp.float32),
            pltpu.SemaphoreType.DMA((_NCH,)),
            pltpu.SemaphoreType.DMA((_NCH,)),
        ],
    )(P)
